# baseline (speedup 1.0000x reference)
import functools

import jax
import jax.numpy as jnp
from jax import lax
from jax.experimental import pallas as pl
from jax.experimental.pallas import tpu as pltpu
from jax.experimental.pallas import tpu_sc as plsc

E = 64
TOPK = 2
H = 768
FF = 1024
T = 2048
P = T * TOPK
TILE = 128
ALIGN = 8
NTOT = P + E * ALIGN
PAD = TILE

NC, NS = 2, 16
NW = NC * NS
CHUNK = P // NW
CT = T // NW
LANES = 16


def _router_body(flat_ref, rw_ref, pos_ref, prob_ref, starts_ref, counts_ref,
                 oh_ref):
    flat = flat_ref[...]
    rw = rw_ref[...]
    logits = lax.dot_general(flat, rw, (((1,), (1,)), ((), ())),
                             preferred_element_type=jnp.float32)
    m = jnp.max(logits, axis=-1, keepdims=True)
    ex = jnp.exp(logits - m)
    probs = ex / jnp.sum(ex, axis=-1, keepdims=True)

    lane = lax.broadcasted_iota(jnp.int32, (T, E), 1)
    v1 = jnp.max(probs, axis=-1, keepdims=True)
    i1 = jnp.min(jnp.where(probs == v1, lane, E), axis=-1, keepdims=True)
    masked = jnp.where(lane == i1, -jnp.inf, probs)
    v2 = jnp.max(masked, axis=-1, keepdims=True)
    i2 = jnp.min(jnp.where(masked == v2, lane, E), axis=-1, keepdims=True)
    tot = v1 + v2
    prob_ref[0:T, :] = jnp.broadcast_to(v1 / tot, (T, LANES))
    prob_ref[T:P, :] = jnp.broadcast_to(v2 / tot, (T, LANES))

    oh_ref[0:T, :] = (lane == i1).astype(jnp.float32)
    oh_ref[T:P, :] = (lane == i2).astype(jnp.float32)

    counts = jnp.sum(oh_ref[...], axis=0, keepdims=True)
    aligned = jnp.floor((counts + (ALIGN - 1)) / ALIGN) * ALIGN
    ltri_e = (lax.broadcasted_iota(jnp.int32, (E, E), 0)
              < lax.broadcasted_iota(jnp.int32, (E, E), 1)).astype(jnp.float32)
    offs = lax.dot_general(aligned, ltri_e, (((1,), (0,)), ((), ())),
                           preferred_element_type=jnp.float32)
    starts_ref[...] = offs.astype(jnp.int32)
    counts_ref[...] = counts.astype(jnp.int32)

    csb = 512
    ltri_t = (lax.broadcasted_iota(jnp.int32, (csb, csb), 0)
              > lax.broadcasted_iota(jnp.int32, (csb, csb), 1)).astype(jnp.float32)

    def block(b, carry):
        ohb = oh_ref[pl.ds(b * csb, csb), :]
        csum_excl = lax.dot_general(ltri_t, ohb, (((1,), (0,)), ((), ())),
                                    preferred_element_type=jnp.float32) + carry
        posb = jnp.sum(ohb * (csum_excl + offs), axis=-1, keepdims=True)
        pos_ref[pl.ds(b * csb, csb), :] = posb.astype(jnp.int32)
        return carry + jnp.sum(ohb, axis=0, keepdims=True)

    lax.fori_loop(0, P // csb, block, jnp.zeros((1, E), jnp.float32))


def _router_call(flat, rw):
    return pl.pallas_call(
        _router_body,
        out_shape=[
            jax.ShapeDtypeStruct((P, 1), jnp.int32),
            jax.ShapeDtypeStruct((P, LANES), jnp.float32),
            jax.ShapeDtypeStruct((1, E), jnp.int32),
            jax.ShapeDtypeStruct((1, E), jnp.int32),
        ],
        scratch_shapes=[pltpu.VMEM((P, E), jnp.float32)],
    )(flat, rw)


HC = CHUNK // 2


def _dispatch_body(flat_hbm, pos_hbm, x_out,
                   pos_v0, pos_v1, rows_v0, rows_v1,
                   s1, s2, s3, s4):
    w = lax.axis_index("s") * NC + lax.axis_index("c")
    base = w * CHUNK
    tok0 = lax.rem(base, T)
    c_p0 = pltpu.async_copy(pos_hbm.at[pl.ds(base, HC)], pos_v0, s1)
    c_r0 = pltpu.async_copy(flat_hbm.at[pl.ds(tok0, HC), :], rows_v0, s2)
    c_p1 = pltpu.async_copy(pos_hbm.at[pl.ds(base + HC, HC)], pos_v1, s3)
    c_r1 = pltpu.async_copy(flat_hbm.at[pl.ds(tok0 + HC, HC), :], rows_v1, s4)
    c_p0.wait()
    c_r0.wait()
    w_r0 = pltpu.async_copy(rows_v0, x_out.at[pos_v0], s2)
    c_p1.wait()
    c_r1.wait()
    w_r1 = pltpu.async_copy(rows_v1, x_out.at[pos_v1], s4)
    w_r0.wait()
    w_r1.wait()


@functools.lru_cache(maxsize=None)
def _sc_mesh():
    return plsc.VectorSubcoreMesh(core_axis_name="c", subcore_axis_name="s",
                                  num_cores=NC, num_subcores=NS)


@functools.lru_cache(maxsize=None)
def _dispatch_kernel():
    return pl.kernel(
        _dispatch_body,
        out_type=jax.ShapeDtypeStruct((NTOT + PAD, H), jnp.float32),
        mesh=_sc_mesh(),
        scratch_types=[
            pltpu.VMEM((HC,), jnp.int32),
            pltpu.VMEM((HC,), jnp.int32),
            pltpu.VMEM((HC, H), jnp.float32),
            pltpu.VMEM((HC, H), jnp.float32),
        ] + [pltpu.SemaphoreType.DMA] * 4,
    )


NBUF = 3


def _ffn_body(starts_ref, counts_ref, x_ref, wg_hbm, wu_hbm, wd_hbm, y_ref,
              wgb, wub, wdb, sg, su, sd):
    e = pl.program_id(0)
    start = pl.multiple_of(starts_ref[0, e], ALIGN)
    count = counts_ref[0, e]

    def issue_gu(i, slot):
        pltpu.make_async_copy(wg_hbm.at[i], wgb.at[slot], sg.at[slot]).start()
        pltpu.make_async_copy(wu_hbm.at[i], wub.at[slot], su.at[slot]).start()

    def issue_d(i, slot):
        pltpu.make_async_copy(wd_hbm.at[i], wdb.at[slot], sd.at[slot]).start()

    @pl.when(e == 0)
    def _prime():
        for k in range(NBUF - 1):
            issue_gu(k, k)
        issue_d(0, 0)

    slot = lax.rem(e, NBUF)
    slot2 = lax.rem(e, 2)

    @pl.when(e + NBUF - 1 < E)
    def _ahead_gu():
        issue_gu(e + NBUF - 1, lax.rem(e + NBUF - 1, NBUF))

    @pl.when(e + 1 < E)
    def _ahead_d():
        issue_d(e + 1, lax.rem(e + 1, 2))

    pltpu.make_async_copy(wg_hbm.at[e], wgb.at[slot], sg.at[slot]).wait()
    pltpu.make_async_copy(wu_hbm.at[e], wub.at[slot], su.at[slot]).wait()
    pltpu.make_async_copy(wd_hbm.at[e], wdb.at[slot2], sd.at[slot2]).wait()

    wg16 = wgb[slot].astype(jnp.bfloat16)
    wu16 = wub[slot].astype(jnp.bfloat16)
    wd16 = wdb[slot2].astype(jnp.bfloat16)

    def tile_body(i, _):
        base = start + i * TILE
        x = x_ref[pl.ds(base, TILE), :].astype(jnp.bfloat16)
        g = lax.dot_general(x, wg16, (((1,), (1,)), ((), ())),
                            preferred_element_type=jnp.float32)
        u = lax.dot_general(x, wu16, (((1,), (1,)), ((), ())),
                            preferred_element_type=jnp.float32)
        act = (g * lax.logistic(g) * u).astype(jnp.bfloat16)
        y = lax.dot_general(act, wd16, (((1,), (1,)), ((), ())),
                            preferred_element_type=jnp.float32)
        y_ref[pl.ds(base, TILE), :] = y
        return 0

    lax.fori_loop(0, (count + TILE - 1) // TILE, tile_body, 0)


def _ffn_call(starts, counts, x_sorted, wg, wu, wd):
    grid_spec = pltpu.PrefetchScalarGridSpec(
        num_scalar_prefetch=2,
        grid=(E,),
        in_specs=[
            pl.BlockSpec((NTOT + PAD, H), lambda e, s, c: (0, 0)),
            pl.BlockSpec(memory_space=pl.ANY),
            pl.BlockSpec(memory_space=pl.ANY),
            pl.BlockSpec(memory_space=pl.ANY),
        ],
        out_specs=pl.BlockSpec((NTOT + PAD, H), lambda e, s, c: (0, 0)),
        scratch_shapes=[
            pltpu.VMEM((NBUF, FF, H), jnp.float32),
            pltpu.VMEM((NBUF, FF, H), jnp.float32),
            pltpu.VMEM((2, H, FF), jnp.float32),
            pltpu.SemaphoreType.DMA((NBUF,)),
            pltpu.SemaphoreType.DMA((NBUF,)),
            pltpu.SemaphoreType.DMA((2,)),
        ],
    )
    return pl.pallas_call(
        _ffn_body,
        grid_spec=grid_spec,
        out_shape=jax.ShapeDtypeStruct((NTOT + PAD, H), jnp.float32),
        compiler_params=pltpu.CompilerParams(
            vmem_limit_bytes=100 * 1024 * 1024),
    )(starts, counts, x_sorted, wg, wu, wd)


HT = CT // 2


def _combine_body(y_hbm, pos_hbm, prob_hbm, out_hbm, p00, p01, p10, p11,
                  q00, q01, q10, q11, a0, a1, b0, b1,
                  s1, s2, s3, s4, s5, s6, s7, s8):
    w = lax.axis_index("s") * NC + lax.axis_index("c")
    t0 = w * CT
    i00 = pltpu.async_copy(pos_hbm.at[pl.ds(t0, HT)], p00, s1)
    i10 = pltpu.async_copy(pos_hbm.at[pl.ds(T + t0, HT)], p10, s2)
    i01 = pltpu.async_copy(pos_hbm.at[pl.ds(t0 + HT, HT)], p01, s3)
    i11 = pltpu.async_copy(pos_hbm.at[pl.ds(T + t0 + HT, HT)], p11, s4)
    j00 = pltpu.async_copy(prob_hbm.at[pl.ds(t0, HT), :], q00, s5)
    j10 = pltpu.async_copy(prob_hbm.at[pl.ds(T + t0, HT), :], q10, s6)
    j01 = pltpu.async_copy(prob_hbm.at[pl.ds(t0 + HT, HT), :], q01, s7)
    j11 = pltpu.async_copy(prob_hbm.at[pl.ds(T + t0 + HT, HT), :], q11, s8)
    i00.wait()
    g00 = pltpu.async_copy(y_hbm.at[p00], a0, s1)
    i10.wait()
    g10 = pltpu.async_copy(y_hbm.at[p10], b0, s2)
    i01.wait()
    g01 = pltpu.async_copy(y_hbm.at[p01], a1, s3)
    i11.wait()
    g11 = pltpu.async_copy(y_hbm.at[p11], b1, s4)

    def add_rows(av, bv, qa, qb):
        def row(r, _):
            pa = qa[r, :]
            pb = qb[r, :]
            for cc in range(H // LANES):
                sl = pl.ds(cc * LANES, LANES)
                av[r, sl] = av[r, sl] * pa + bv[r, sl] * pb
            return 0
        lax.fori_loop(0, HT, row, 0)

    g00.wait()
    g10.wait()
    j00.wait()
    j10.wait()
    add_rows(a0, b0, q00, q10)
    o0 = pltpu.async_copy(a0, out_hbm.at[pl.ds(t0, HT), :], s1)
    g01.wait()
    g11.wait()
    j01.wait()
    j11.wait()
    add_rows(a1, b1, q01, q11)
    o1 = pltpu.async_copy(a1, out_hbm.at[pl.ds(t0 + HT, HT), :], s2)
    o0.wait()
    o1.wait()


@functools.lru_cache(maxsize=None)
def _combine_kernel():
    return pl.kernel(
        _combine_body,
        out_type=jax.ShapeDtypeStruct((T, H), jnp.float32),
        mesh=_sc_mesh(),
        scratch_types=[
            pltpu.VMEM((HT,), jnp.int32),
            pltpu.VMEM((HT,), jnp.int32),
            pltpu.VMEM((HT,), jnp.int32),
            pltpu.VMEM((HT,), jnp.int32),
            pltpu.VMEM((HT, LANES), jnp.float32),
            pltpu.VMEM((HT, LANES), jnp.float32),
            pltpu.VMEM((HT, LANES), jnp.float32),
            pltpu.VMEM((HT, LANES), jnp.float32),
            pltpu.VMEM((HT, H), jnp.float32),
            pltpu.VMEM((HT, H), jnp.float32),
            pltpu.VMEM((HT, H), jnp.float32),
            pltpu.VMEM((HT, H), jnp.float32),
        ] + [pltpu.SemaphoreType.DMA] * 8,
    )


def kernel(hidden_states, router_weight, Wg, Wu, Wd):
    b, s, h = hidden_states.shape
    flat = hidden_states.reshape(T, H)
    pos2, prob2, starts, counts = _router_call(flat, router_weight)
    pos = pos2.reshape(P)
    x_sorted = _dispatch_kernel()(flat, pos)
    y = _ffn_call(starts, counts, x_sorted, Wg, Wu, Wd)
    out = _combine_kernel()(y, pos, prob2)
    return out.reshape(b, s, h)


# speedup vs baseline: 1.0276x; 1.0276x over previous
import functools

import jax
import jax.numpy as jnp
from jax import lax
from jax.experimental import pallas as pl
from jax.experimental.pallas import tpu as pltpu
from jax.experimental.pallas import tpu_sc as plsc

E = 64
TOPK = 2
H = 768
FF = 1024
T = 2048
P = T * TOPK
TILE = 128
ALIGN = 8
NTOT = P + E * ALIGN
PAD = TILE

NC, NS = 2, 16
NW = NC * NS
CHUNK = P // NW
CT = T // NW
LANES = 16


def _router_body(flat_ref, rw_ref, pos_ref, prob_ref, starts_ref, counts_ref,
                 oh_ref):
    flat = flat_ref[...]
    rw = rw_ref[...]
    logits = lax.dot_general(flat, rw, (((1,), (1,)), ((), ())),
                             preferred_element_type=jnp.float32)
    m = jnp.max(logits, axis=-1, keepdims=True)
    ex = jnp.exp(logits - m)
    probs = ex / jnp.sum(ex, axis=-1, keepdims=True)

    lane = lax.broadcasted_iota(jnp.int32, (T, E), 1)
    v1 = jnp.max(probs, axis=-1, keepdims=True)
    i1 = jnp.min(jnp.where(probs == v1, lane, E), axis=-1, keepdims=True)
    masked = jnp.where(lane == i1, -jnp.inf, probs)
    v2 = jnp.max(masked, axis=-1, keepdims=True)
    i2 = jnp.min(jnp.where(masked == v2, lane, E), axis=-1, keepdims=True)
    tot = v1 + v2
    prob_ref[0:T, :] = jnp.broadcast_to(v1 / tot, (T, LANES))
    prob_ref[T:P, :] = jnp.broadcast_to(v2 / tot, (T, LANES))

    oh_ref[0:T, :] = (lane == i1).astype(jnp.float32)
    oh_ref[T:P, :] = (lane == i2).astype(jnp.float32)

    counts = jnp.sum(oh_ref[...], axis=0, keepdims=True)
    aligned = jnp.floor((counts + (ALIGN - 1)) / ALIGN) * ALIGN
    ltri_e = (lax.broadcasted_iota(jnp.int32, (E, E), 0)
              < lax.broadcasted_iota(jnp.int32, (E, E), 1)).astype(jnp.float32)
    offs = lax.dot_general(aligned, ltri_e, (((1,), (0,)), ((), ())),
                           preferred_element_type=jnp.float32)
    starts_ref[...] = offs.astype(jnp.int32)
    counts_ref[...] = counts.astype(jnp.int32)

    csb = 512
    ltri_t = (lax.broadcasted_iota(jnp.int32, (csb, csb), 0)
              > lax.broadcasted_iota(jnp.int32, (csb, csb), 1)).astype(jnp.float32)

    def block(b, carry):
        ohb = oh_ref[pl.ds(b * csb, csb), :]
        csum_excl = lax.dot_general(ltri_t, ohb, (((1,), (0,)), ((), ())),
                                    preferred_element_type=jnp.float32) + carry
        posb = jnp.sum(ohb * (csum_excl + offs), axis=-1, keepdims=True)
        pos_ref[pl.ds(b * csb, csb), :] = posb.astype(jnp.int32)
        return carry + jnp.sum(ohb, axis=0, keepdims=True)

    lax.fori_loop(0, P // csb, block, jnp.zeros((1, E), jnp.float32))


def _router_call(flat, rw):
    return pl.pallas_call(
        _router_body,
        out_shape=[
            jax.ShapeDtypeStruct((P, 1), jnp.int32),
            jax.ShapeDtypeStruct((P, LANES), jnp.float32),
            jax.ShapeDtypeStruct((1, E), jnp.int32),
            jax.ShapeDtypeStruct((1, E), jnp.int32),
        ],
        scratch_shapes=[pltpu.VMEM((P, E), jnp.float32)],
    )(flat, rw)


HC = CHUNK // 2


def _dispatch_body(flat_hbm, pos_hbm, x_out,
                   pos_v0, pos_v1, rows_v0, rows_v1,
                   s1, s2, s3, s4):
    w = lax.axis_index("s") * NC + lax.axis_index("c")
    base = w * CHUNK
    tok0 = lax.rem(base, T)
    c_p0 = pltpu.async_copy(pos_hbm.at[pl.ds(base, HC)], pos_v0, s1)
    c_r0 = pltpu.async_copy(flat_hbm.at[pl.ds(tok0, HC), :], rows_v0, s2)
    c_p1 = pltpu.async_copy(pos_hbm.at[pl.ds(base + HC, HC)], pos_v1, s3)
    c_r1 = pltpu.async_copy(flat_hbm.at[pl.ds(tok0 + HC, HC), :], rows_v1, s4)
    c_p0.wait()
    c_r0.wait()
    w_r0 = pltpu.async_copy(rows_v0, x_out.at[pos_v0], s2)
    c_p1.wait()
    c_r1.wait()
    w_r1 = pltpu.async_copy(rows_v1, x_out.at[pos_v1], s4)
    w_r0.wait()
    w_r1.wait()


@functools.lru_cache(maxsize=None)
def _sc_mesh():
    return plsc.VectorSubcoreMesh(core_axis_name="c", subcore_axis_name="s",
                                  num_cores=NC, num_subcores=NS)


@functools.lru_cache(maxsize=None)
def _dispatch_kernel():
    return pl.kernel(
        _dispatch_body,
        out_type=jax.ShapeDtypeStruct((NTOT + PAD, H), jnp.float32),
        mesh=_sc_mesh(),
        scratch_types=[
            pltpu.VMEM((HC,), jnp.int32),
            pltpu.VMEM((HC,), jnp.int32),
            pltpu.VMEM((HC, H), jnp.float32),
            pltpu.VMEM((HC, H), jnp.float32),
        ] + [pltpu.SemaphoreType.DMA] * 4,
    )


def _ffn_body(starts_ref, counts_ref, x_ref, wg_ref, wu_ref, wd_ref,
              y_ref):
    e = pl.program_id(0)
    start = pl.multiple_of(starts_ref[0, e], ALIGN)
    count = counts_ref[0, e]
    wg = wg_ref[0]
    wu = wu_ref[0]
    wd = wd_ref[0]

    wg16 = wg.astype(jnp.bfloat16)
    wu16 = wu.astype(jnp.bfloat16)
    wd16 = wd.astype(jnp.bfloat16)

    def tile_body(i, _):
        base = start + i * TILE
        x = x_ref[pl.ds(base, TILE), :].astype(jnp.bfloat16)
        g = lax.dot_general(x, wg16, (((1,), (1,)), ((), ())),
                            preferred_element_type=jnp.float32)
        u = lax.dot_general(x, wu16, (((1,), (1,)), ((), ())),
                            preferred_element_type=jnp.float32)
        act = (g * lax.logistic(g) * u).astype(jnp.bfloat16)
        y = lax.dot_general(act, wd16, (((1,), (1,)), ((), ())),
                            preferred_element_type=jnp.float32)
        y_ref[pl.ds(base, TILE), :] = y
        return 0

    lax.fori_loop(0, (count + TILE - 1) // TILE, tile_body, 0)


def _ffn_call(starts, counts, x_sorted, wg, wu, wd):
    grid_spec = pltpu.PrefetchScalarGridSpec(
        num_scalar_prefetch=2,
        grid=(E,),
        in_specs=[
            pl.BlockSpec((NTOT + PAD, H), lambda e, s, c: (0, 0)),
            pl.BlockSpec((1, FF, H), lambda e, s, c: (e, 0, 0)),
            pl.BlockSpec((1, FF, H), lambda e, s, c: (e, 0, 0)),
            pl.BlockSpec((1, H, FF), lambda e, s, c: (e, 0, 0)),
        ],
        out_specs=pl.BlockSpec((NTOT + PAD, H), lambda e, s, c: (0, 0)),
    )
    return pl.pallas_call(
        _ffn_body,
        grid_spec=grid_spec,
        out_shape=jax.ShapeDtypeStruct((NTOT + PAD, H), jnp.float32),
    )(starts, counts, x_sorted, wg, wu, wd)


HT = CT // 2


def _combine_body(y_hbm, pos_hbm, prob_hbm, out_hbm, p00, p01, p10, p11,
                  q00, q01, q10, q11, a0, a1, b0, b1,
                  s1, s2, s3, s4, s5, s6, s7, s8):
    w = lax.axis_index("s") * NC + lax.axis_index("c")
    t0 = w * CT
    i00 = pltpu.async_copy(pos_hbm.at[pl.ds(t0, HT)], p00, s1)
    i10 = pltpu.async_copy(pos_hbm.at[pl.ds(T + t0, HT)], p10, s2)
    i01 = pltpu.async_copy(pos_hbm.at[pl.ds(t0 + HT, HT)], p01, s3)
    i11 = pltpu.async_copy(pos_hbm.at[pl.ds(T + t0 + HT, HT)], p11, s4)
    j00 = pltpu.async_copy(prob_hbm.at[pl.ds(t0, HT), :], q00, s5)
    j10 = pltpu.async_copy(prob_hbm.at[pl.ds(T + t0, HT), :], q10, s6)
    j01 = pltpu.async_copy(prob_hbm.at[pl.ds(t0 + HT, HT), :], q01, s7)
    j11 = pltpu.async_copy(prob_hbm.at[pl.ds(T + t0 + HT, HT), :], q11, s8)
    i00.wait()
    g00 = pltpu.async_copy(y_hbm.at[p00], a0, s1)
    i10.wait()
    g10 = pltpu.async_copy(y_hbm.at[p10], b0, s2)
    i01.wait()
    g01 = pltpu.async_copy(y_hbm.at[p01], a1, s3)
    i11.wait()
    g11 = pltpu.async_copy(y_hbm.at[p11], b1, s4)

    def add_rows(av, bv, qa, qb):
        def row(r, _):
            pa = qa[r, :]
            pb = qb[r, :]
            for cc in range(H // LANES):
                sl = pl.ds(cc * LANES, LANES)
                av[r, sl] = av[r, sl] * pa + bv[r, sl] * pb
            return 0
        lax.fori_loop(0, HT, row, 0)

    g00.wait()
    g10.wait()
    j00.wait()
    j10.wait()
    add_rows(a0, b0, q00, q10)
    o0 = pltpu.async_copy(a0, out_hbm.at[pl.ds(t0, HT), :], s1)
    g01.wait()
    g11.wait()
    j01.wait()
    j11.wait()
    add_rows(a1, b1, q01, q11)
    o1 = pltpu.async_copy(a1, out_hbm.at[pl.ds(t0 + HT, HT), :], s2)
    o0.wait()
    o1.wait()


@functools.lru_cache(maxsize=None)
def _combine_kernel():
    return pl.kernel(
        _combine_body,
        out_type=jax.ShapeDtypeStruct((T, H), jnp.float32),
        mesh=_sc_mesh(),
        scratch_types=[
            pltpu.VMEM((HT,), jnp.int32),
            pltpu.VMEM((HT,), jnp.int32),
            pltpu.VMEM((HT,), jnp.int32),
            pltpu.VMEM((HT,), jnp.int32),
            pltpu.VMEM((HT, LANES), jnp.float32),
            pltpu.VMEM((HT, LANES), jnp.float32),
            pltpu.VMEM((HT, LANES), jnp.float32),
            pltpu.VMEM((HT, LANES), jnp.float32),
            pltpu.VMEM((HT, H), jnp.float32),
            pltpu.VMEM((HT, H), jnp.float32),
            pltpu.VMEM((HT, H), jnp.float32),
            pltpu.VMEM((HT, H), jnp.float32),
        ] + [pltpu.SemaphoreType.DMA] * 8,
    )


def kernel(hidden_states, router_weight, Wg, Wu, Wd):
    b, s, h = hidden_states.shape
    flat = hidden_states.reshape(T, H)
    pos2, prob2, starts, counts = _router_call(flat, router_weight)
    pos = pos2.reshape(P)
    x_sorted = _dispatch_kernel()(flat, pos)
    y = _ffn_call(starts, counts, x_sorted, Wg, Wu, Wd)
    out = _combine_kernel()(y, pos, prob2)
    return out.reshape(b, s, h)
